# Initial kernel scaffold; baseline (speedup 1.0000x reference)
import jax
import jax.numpy as jnp
from jax.experimental import pallas as pl


def kernel(features, edge_index, e_weight, b1, b2, b3, W1, bd1, W2, bd2, W3, bd3):
    raise NotImplementedError("write your pallas kernel here")


# speedup vs baseline: 5.8582x; 5.8582x over previous
import jax
import jax.numpy as jnp
from jax import lax
from jax.experimental import pallas as pl
from jax.experimental.pallas import tpu as pltpu
from jax.experimental.pallas import tpu_sc as plsc

N = 10000
E = 320000
D = 128

NC = 2
NS = 16
L = 16

N_PAD = 10240
ROWS_PER_TILE = N_PAD // NS
E_PAD = 327680
EROWS = E_PAD // 128
AROWS = EROWS // NS
BROWS = EROWS // (NC * NS)


def _rsqrt16(x):
    i = lax.bitcast_convert_type(x, jnp.int32)
    i = jnp.int32(0x5F3759DF) - lax.shift_right_logical(i, 1)
    y = lax.bitcast_convert_type(i, jnp.float32)
    for _ in range(3):
        y = y * (1.5 - 0.5 * x * y * y)
    return y


def _sc_body(feat_hbm, src_hbm, dst_hbm, w_hbm, part_hbm, ideg_hbm,
             agg_sh, odeg_sh, ideg_sh, onorm_sh,
             idx_v, dst_v, w_v, rows_v, nrm_v, sem):
    c = lax.axis_index("c")
    s = lax.axis_index("s")
    r0 = s * ROWS_PER_TILE

    zeros16 = jnp.zeros((L,), jnp.float32)
    ones16 = jnp.ones((L,), jnp.float32)

    def zero_row(i, _):
        for k in range(D // L):
            rows_v[i, pl.ds(k * L, L)] = zeros16
        return 0

    lax.fori_loop(0, 128, zero_row, 0)
    for k in range(128 // L):
        nrm_v[0, pl.ds(k * L, L)] = ones16
        nrm_v[1, pl.ds(k * L, L)] = zeros16
    for q in range(ROWS_PER_TILE // 128):
        pltpu.sync_copy(nrm_v.at[1], odeg_sh.at[pl.ds(r0 + q * 128, 128)])
        pltpu.sync_copy(nrm_v.at[1], ideg_sh.at[pl.ds(r0 + q * 128, 128)])
        pltpu.sync_copy(rows_v, agg_sh.at[pl.ds(r0 + q * 128, 128)])
    plsc.subcore_barrier()

    def count_degrees(edges_hbm, row0, deg_target):
        pltpu.sync_copy(edges_hbm.at[pl.ds(row0, BROWS)], idx_v)
        pending = []
        for j in range(BROWS):
            pending.append(
                pltpu.async_copy(nrm_v.at[0], deg_target.at[idx_v.at[j]],
                                 sem, add=True))
            if len(pending) == 8:
                for p in pending:
                    p.wait()
                pending = []
        for p in pending:
            p.wait()

    baseA = s * AROWS
    count_degrees(src_hbm, baseA, odeg_sh)
    count_degrees(src_hbm, baseA + BROWS, odeg_sh)
    count_degrees(dst_hbm, baseA, ideg_sh)
    count_degrees(dst_hbm, baseA + BROWS, ideg_sh)
    plsc.subcore_barrier()

    for q in range(ROWS_PER_TILE // 128):
        pltpu.sync_copy(odeg_sh.at[pl.ds(r0 + q * 128, 128)], nrm_v.at[1])
        for k in range(128 // L):
            x = jnp.maximum(nrm_v[1, pl.ds(k * L, L)], 1.0)
            nrm_v[1, pl.ds(k * L, L)] = _rsqrt16(x)
        pltpu.sync_copy(nrm_v.at[1], onorm_sh.at[pl.ds(r0 + q * 128, 128)])

    @pl.when(c == 0)
    def _():
        pltpu.sync_copy(ideg_sh.at[pl.ds(r0, ROWS_PER_TILE)],
                        ideg_hbm.at[pl.ds(r0, ROWS_PER_TILE)])

    plsc.subcore_barrier()

    baseB = (c * NS + s) * BROWS
    pltpu.sync_copy(src_hbm.at[pl.ds(baseB, BROWS)], idx_v)
    pltpu.sync_copy(w_hbm.at[pl.ds(baseB, BROWS)], w_v)
    for j0 in range(0, BROWS, 4):
        pending = [
            pltpu.async_copy(onorm_sh.at[idx_v.at[j0 + jj]], nrm_v.at[jj],
                             sem)
            for jj in range(4)
        ]
        for p in pending:
            p.wait()
        for jj in range(4):
            for k in range(128 // L):
                w_v[j0 + jj, pl.ds(k * L, L)] = (
                    w_v[j0 + jj, pl.ds(k * L, L)]
                    * nrm_v[jj, pl.ds(k * L, L)])

    def group(g, _):
        pltpu.sync_copy(dst_hbm.at[pl.ds(baseB + g * 8, 8)], dst_v)

        def chunk(jj, _):
            j = g * 8 + jj
            pltpu.async_copy(feat_hbm.at[idx_v.at[j]], rows_v, sem).wait()

            def scale_group(gg, _):
                wp16 = w_v[j, pl.ds(gg * L, L)]
                for di in range(L):
                    sc = wp16[di]
                    i = gg * L + di
                    for k in range(D // L):
                        rows_v[i, pl.ds(k * L, L)] = (
                            rows_v[i, pl.ds(k * L, L)] * sc)
                return 0

            lax.fori_loop(0, 128 // L, scale_group, 0)
            pltpu.sync_copy(rows_v, agg_sh.at[dst_v.at[jj]], add=True)
            return 0

        lax.fori_loop(0, 8, chunk, 0)
        return 0

    lax.fori_loop(0, BROWS // 8, group, 0)
    plsc.subcore_barrier()

    pltpu.sync_copy(agg_sh.at[pl.ds(r0, ROWS_PER_TILE)],
                    part_hbm.at[c, pl.ds(r0, ROWS_PER_TILE)])


def _sc_aggregate(feat_p, src2, dst2, w2):
    mesh = plsc.VectorSubcoreMesh(core_axis_name="c", subcore_axis_name="s",
                                  num_cores=NC, num_subcores=NS)
    return pl.kernel(
        _sc_body,
        out_type=(
            jax.ShapeDtypeStruct((NC, N_PAD, D), jnp.float32),
            jax.ShapeDtypeStruct((N_PAD,), jnp.float32),
        ),
        mesh=mesh,
        scratch_types=[
            pltpu.VMEM_SHARED((N_PAD, D), jnp.float32),
            pltpu.VMEM_SHARED((N_PAD,), jnp.float32),
            pltpu.VMEM_SHARED((N_PAD,), jnp.float32),
            pltpu.VMEM_SHARED((N_PAD,), jnp.float32),
            pltpu.VMEM((BROWS, 128), jnp.int32),
            pltpu.VMEM((8, 128), jnp.int32),
            pltpu.VMEM((BROWS, 128), jnp.float32),
            pltpu.VMEM((128, D), jnp.float32),
            pltpu.VMEM((4, 128), jnp.float32),
            pltpu.SemaphoreType.DMA,
        ],
        name="gcn_sc_aggregate",
    )(feat_p, src2, dst2, w2)


ROWS_BLK = 256
N_BLKS = N_PAD // ROWS_BLK


def _tc_tail_body(pa_ref, ideg_ref, b3_ref, w1_ref, bd1_ref, w2_ref, bd2_ref,
                  w3_ref, bd3_ref, out_ref, acc_ref):
    i = pl.program_id(0)

    @pl.when(i == 0)
    def _():
        acc_ref[...] = jnp.zeros_like(acc_ref)

    agg = pa_ref[0] + pa_ref[1]
    ideg = jnp.maximum(ideg_ref[...], 1.0)
    inorm = lax.rsqrt(ideg)
    h = jnp.maximum(agg * inorm + b3_ref[...], 0.0)
    row = i * ROWS_BLK + lax.broadcasted_iota(jnp.int32, (ROWS_BLK, 1), 0)
    h = jnp.where(row < N, h, 0.0)
    acc_ref[...] += jnp.sum(h, axis=0, keepdims=True)

    @pl.when(i == N_BLKS - 1)
    def _():
        hg = acc_ref[...] * (1.0 / N)
        m = jnp.maximum(jnp.dot(hg, w1_ref[...],
                                preferred_element_type=jnp.float32)
                        + bd1_ref[...], 0.0)
        m = jnp.maximum(jnp.dot(m, w2_ref[...],
                                preferred_element_type=jnp.float32)
                        + bd2_ref[...], 0.0)
        z = jnp.dot(m, w3_ref[...],
                    preferred_element_type=jnp.float32) + bd3_ref[...]
        out_ref[...] = 1.0 / (1.0 + jnp.exp(-z))


def _tc_tail(part, ideg2, b3, W1, bd1, W2, bd2, W3, bd3):
    full = lambda shape: pl.BlockSpec(shape, lambda i: tuple(0 for _ in shape))
    return pl.pallas_call(
        _tc_tail_body,
        grid=(N_BLKS,),
        in_specs=[
            pl.BlockSpec((NC, ROWS_BLK, D), lambda i: (0, i, 0)),
            pl.BlockSpec((ROWS_BLK, 1), lambda i: (i, 0)),
            full((1, D)),
            full((D, 16)), full((1, 16)),
            full((16, 8)), full((1, 8)),
            full((8, 1)), full((1, 1)),
        ],
        out_specs=pl.BlockSpec((1, 1), lambda i: (0, 0)),
        out_shape=jax.ShapeDtypeStruct((1, 1), jnp.float32),
        scratch_shapes=[pltpu.VMEM((1, D), jnp.float32)],
    )(part, ideg2, b3, W1, bd1, W2, bd2, W3, bd3)


def kernel(features, edge_index, e_weight, b1, b2, b3, W1, bd1, W2, bd2, W3,
           bd3):
    del b1, b2
    src = edge_index[0].astype(jnp.int32)
    dst = edge_index[1].astype(jnp.int32)
    w = e_weight.astype(jnp.float32)

    npad = E_PAD - E
    pad_idx = (N + (jnp.arange(npad, dtype=jnp.int32) % (N_PAD - N)))
    src2 = jnp.concatenate([src, pad_idx]).reshape(EROWS, 128)
    dst2 = jnp.concatenate([dst, pad_idx]).reshape(EROWS, 128)
    w2 = jnp.concatenate([w, jnp.zeros((npad,), jnp.float32)]).reshape(
        EROWS, 128)
    feat_p = jnp.pad(features, ((0, N_PAD - N), (0, 0)))

    part, ideg = _sc_aggregate(feat_p, src2, dst2, w2)
    return _tc_tail(part, ideg.reshape(N_PAD, 1), b3.reshape(1, D),
                    W1, bd1.reshape(1, 16), W2, bd2.reshape(1, 8),
                    W3, bd3.reshape(1, 1))
